# speedup vs baseline: 3.4058x; 3.4058x over previous
import functools

import jax
import jax.numpy as jnp
from jax import lax
from jax.experimental import pallas as pl
from jax.experimental.pallas import tpu as pltpu
from jax.experimental.pallas import tpu_sc as plsc

_HIDDEN = 128
_LANES = 16
_NW = 32
_K = 128


def _emb_body(seq_hbm, cidx_hbm, tok_hbm, psum_hbm, out_hbm,
              idx_v, cidx_v, tok_b, psum_b, sem_t, sem_p):
    n = out_hbm.shape[0]
    rows_per_w = n // _NW
    nchunk = rows_per_w // _K
    wid = lax.axis_index("s") * 2 + lax.axis_index("c")
    wbase = wid * rows_per_w

    def chunk(c, carry):
        base = wbase + c * _K
        pltpu.sync_copy(seq_hbm.at[pl.ds(base, _K)], idx_v)
        pltpu.sync_copy(cidx_hbm.at[pl.ds(base, _K)], cidx_v)
        cp_t = pltpu.async_copy(tok_hbm.at[idx_v], tok_b, sem_t)
        cp_p = pltpu.async_copy(psum_hbm.at[cidx_v], psum_b, sem_p)
        cp_t.wait()
        cp_p.wait()

        def row(r, carry2):
            for j in range(_HIDDEN // _LANES):
                sl = pl.ds(j * _LANES, _LANES)
                tok_b[r, sl] = tok_b[r, sl] + psum_b[r, sl]
            return carry2

        lax.fori_loop(0, _K, row, 0, unroll=4)
        pltpu.sync_copy(tok_b, out_hbm.at[pl.ds(base, _K)])
        return carry

    lax.fori_loop(0, nchunk, chunk, 0)


def kernel(seq, seg, token_table, position_table, segment_table):
    batch, sent = seq.shape
    n = batch * sent
    seq_flat = seq.reshape(n).astype(jnp.int32)
    cidx = (2 * jnp.arange(sent, dtype=jnp.int32)[None, :]
            + seg.astype(jnp.int32)).reshape(n)
    psum = (position_table[:, None, :] + segment_table[None, :, :]
            ).reshape(2 * sent, _HIDDEN)

    mesh = plsc.VectorSubcoreMesh(core_axis_name="c", subcore_axis_name="s")
    run = functools.partial(
        pl.kernel,
        out_type=jax.ShapeDtypeStruct((n, _HIDDEN), jnp.float32),
        mesh=mesh,
        scratch_types=[
            pltpu.VMEM((_K,), jnp.int32),
            pltpu.VMEM((_K,), jnp.int32),
            pltpu.VMEM((_K, _HIDDEN), jnp.float32),
            pltpu.VMEM((_K, _HIDDEN), jnp.float32),
            pltpu.SemaphoreType.DMA,
            pltpu.SemaphoreType.DMA,
        ],
    )(_emb_body)
    out = run(seq_flat, cidx, token_table, psum)
    return out.reshape(batch, sent, _HIDDEN)


# speedup vs baseline: 5.8422x; 1.7154x over previous
import functools

import jax
import jax.numpy as jnp
from jax import lax
from jax.experimental import pallas as pl
from jax.experimental.pallas import tpu as pltpu
from jax.experimental.pallas import tpu_sc as plsc

_HIDDEN = 128
_LANES = 16
_NW = 32
_K = 128


def _emb_body(seq_hbm, cidx_hbm, tok_hbm, psum_hbm, out_hbm,
              idx_v, cidx_v, tok_b, psum_b, sem_t, sem_p):
    n = out_hbm.shape[0]
    rows_per_w = n // _NW
    nchunk = rows_per_w // _K
    wid = lax.axis_index("s") * 2 + lax.axis_index("c")
    wbase = wid * rows_per_w

    def chunk(c, carry):
        base = wbase + c * _K
        pltpu.sync_copy(seq_hbm.at[pl.ds(base, _K)], idx_v)
        pltpu.sync_copy(cidx_hbm.at[pl.ds(base, _K)], cidx_v)
        cp_p = pltpu.async_copy(psum_hbm.at[cidx_v], tok_b, sem_p)
        cp_p.wait()
        cp_t = pltpu.async_copy(tok_hbm.at[idx_v], tok_b, sem_t, add=True)
        cp_t.wait()
        pltpu.sync_copy(tok_b, out_hbm.at[pl.ds(base, _K)])
        return carry

    lax.fori_loop(0, nchunk, chunk, 0)


def kernel(seq, seg, token_table, position_table, segment_table):
    batch, sent = seq.shape
    n = batch * sent
    seq_flat = seq.reshape(n).astype(jnp.int32)
    cidx = (2 * jnp.arange(sent, dtype=jnp.int32)[None, :]
            + seg.astype(jnp.int32)).reshape(n)
    psum = (position_table[:, None, :] + segment_table[None, :, :]
            ).reshape(2 * sent, _HIDDEN)

    mesh = plsc.VectorSubcoreMesh(core_axis_name="c", subcore_axis_name="s")
    run = functools.partial(
        pl.kernel,
        out_type=jax.ShapeDtypeStruct((n, _HIDDEN), jnp.float32),
        mesh=mesh,
        scratch_types=[
            pltpu.VMEM((_K,), jnp.int32),
            pltpu.VMEM((_K,), jnp.int32),
            pltpu.VMEM((_K, _HIDDEN), jnp.float32),
            pltpu.VMEM((_K, _HIDDEN), jnp.float32),
            pltpu.SemaphoreType.DMA,
            pltpu.SemaphoreType.DMA,
        ],
    )(_emb_body)
    out = run(seq_flat, cidx, token_table, psum)
    return out.reshape(batch, sent, _HIDDEN)


# speedup vs baseline: 10.0414x; 1.7188x over previous
import functools

import jax
import jax.numpy as jnp
from jax import lax
from jax.experimental import pallas as pl
from jax.experimental.pallas import tpu as pltpu
from jax.experimental.pallas import tpu_sc as plsc

_HIDDEN = 128
_NW = 32
_K = 128
_NBUF = 4


def _emb_body(seq_hbm, cidx_hbm, tok_hbm, psum_hbm, out_hbm,
              idx_all, cidx_all, bufs,
              sem_p0, sem_p1, sem_p2, sem_p3, sem_t,
              sem_o0, sem_o1, sem_o2, sem_o3):
    sem_p = [sem_p0, sem_p1, sem_p2, sem_p3]
    sem_o = [sem_o0, sem_o1, sem_o2, sem_o3]
    n = out_hbm.shape[0]
    rows_per_w = n // _NW
    nchunk = rows_per_w // _K
    wid = lax.axis_index("s") * 2 + lax.axis_index("c")
    wbase = wid * rows_per_w

    pltpu.sync_copy(seq_hbm.at[pl.ds(wbase, rows_per_w)], idx_all)
    pltpu.sync_copy(cidx_hbm.at[pl.ds(wbase, rows_per_w)], cidx_all)

    def fire_p(cc, b):
        pltpu.async_copy(
            psum_hbm.at[cidx_all.at[pl.ds(cc * _K, _K)]], bufs.at[b],
            sem_p[b])

    def step(cc, b, first, fire_ahead):
        pltpu.make_async_copy(
            psum_hbm.at[cidx_all.at[pl.ds(cc * _K, _K)]], bufs.at[b],
            sem_p[b]).wait()
        cp_t = pltpu.async_copy(
            tok_hbm.at[idx_all.at[pl.ds(cc * _K, _K)]], bufs.at[b],
            sem_t, add=True)
        if not first:
            b2 = (b - 1) % _NBUF
            pltpu.make_async_copy(
                bufs.at[b2], out_hbm.at[pl.ds(wbase + (cc - 1) * _K, _K)],
                sem_o[b2]).wait()
        if fire_ahead:
            fire_p(cc + 3, (b + 3) % _NBUF)
        cp_t.wait()
        pltpu.async_copy(
            bufs.at[b], out_hbm.at[pl.ds(wbase + cc * _K, _K)], sem_o[b])

    fire_p(0, 0)
    fire_p(1, 1)
    fire_p(2, 2)
    step(0, 0, True, True)
    step(1, 1, False, True)
    step(2, 2, False, True)
    step(3, 3, False, True)

    def group(i, carry):
        for b in range(_NBUF):
            step(i * _NBUF + b, b, False, True)
        return carry

    lax.fori_loop(1, nchunk // _NBUF - 1, group, 0)

    last = nchunk - _NBUF
    step(last + 0, 0, False, True)
    step(last + 1, 1, False, False)
    step(last + 2, 2, False, False)
    step(last + 3, 3, False, False)
    pltpu.make_async_copy(
        bufs.at[_NBUF - 1],
        out_hbm.at[pl.ds(wbase + (nchunk - 1) * _K, _K)],
        sem_o[_NBUF - 1]).wait()


def kernel(seq, seg, token_table, position_table, segment_table):
    batch, sent = seq.shape
    n = batch * sent
    rows_per_w = n // _NW
    seq_flat = seq.reshape(n).astype(jnp.int32)
    cidx = (2 * jnp.arange(sent, dtype=jnp.int32)[None, :]
            + seg.astype(jnp.int32)).reshape(n)
    psum = (position_table[:, None, :] + segment_table[None, :, :]
            ).reshape(2 * sent, _HIDDEN)

    mesh = plsc.VectorSubcoreMesh(core_axis_name="c", subcore_axis_name="s")
    run = functools.partial(
        pl.kernel,
        out_type=jax.ShapeDtypeStruct((n, _HIDDEN), jnp.float32),
        mesh=mesh,
        scratch_types=[
            pltpu.VMEM((rows_per_w,), jnp.int32),
            pltpu.VMEM((rows_per_w,), jnp.int32),
            pltpu.VMEM((_NBUF, _K, _HIDDEN), jnp.float32),
        ] + [pltpu.SemaphoreType.DMA] * 9,
    )(_emb_body)
    out = run(seq_flat, cidx, token_table, psum)
    return out.reshape(batch, sent, _HIDDEN)
